# baseline (speedup 1.0000x reference)
import jax
import jax.numpy as jnp
from jax import lax
from jax.experimental import pallas as pl
from jax.experimental.pallas import tpu as pltpu
from jax.experimental.pallas import tpu_sc as plsc

B = 16384
D = 16
NW = 32
BPW = B // NW
NG = BPW // 16

NROWS = 1000000


def _mf_body(user_hbm, item_hbm, uft_hbm, ift_hbm, out_hbm,
             uidx_v, vidx_v, uoff_v, voff_v, ucols_v, vcols_v, out_v,
             sem_u, sem_v):
    c = lax.axis_index("c")
    s = lax.axis_index("s")
    wid = s * 2 + c
    base = wid * BPW

    pltpu.sync_copy(user_hbm.at[pl.ds(base, BPW)], uidx_v)
    pltpu.sync_copy(item_hbm.at[pl.ds(base, BPW)], vidx_v)

    def build_offsets(g, _):
        gbase = g * 16
        u = uidx_v[pl.ds(gbase, 16)]
        v = vidx_v[pl.ds(gbase, 16)]
        for r in range(D):
            uoff_v[pl.ds(r * BPW + gbase, 16)] = u + r * NROWS
            voff_v[pl.ds(r * BPW + gbase, 16)] = v + r * NROWS
        return ()

    lax.fori_loop(0, NG, build_offsets, ())

    cp_u = pltpu.async_copy(uft_hbm.at[0].at[uoff_v], ucols_v, sem_u)
    cp_v = pltpu.async_copy(ift_hbm.at[0].at[voff_v], vcols_v, sem_v)
    cp_u.wait()
    cp_v.wait()

    def group(g, _):
        gbase = g * 16
        acc = jnp.zeros((16,), jnp.float32)
        for r in range(D):
            acc = acc + (ucols_v[pl.ds(r * BPW + gbase, 16)] *
                         vcols_v[pl.ds(r * BPW + gbase, 16)])
        out_v[pl.ds(gbase, 16)] = acc
        return ()

    lax.fori_loop(0, NG, group, ())

    pltpu.sync_copy(out_v, out_hbm.at[pl.ds(base, BPW)])


def kernel(user, item, user_factors, item_factors):
    mesh = plsc.VectorSubcoreMesh(core_axis_name="c", subcore_axis_name="s")
    k = pl.kernel(
        _mf_body,
        out_type=jax.ShapeDtypeStruct((B,), jnp.float32),
        mesh=mesh,
        compiler_params=pltpu.CompilerParams(
            needs_layout_passes=False, use_tc_tiling_on_sc=False),
        scratch_types=[
            pltpu.VMEM((BPW,), jnp.int32),
            pltpu.VMEM((BPW,), jnp.int32),
            pltpu.VMEM((D * BPW,), jnp.int32),
            pltpu.VMEM((D * BPW,), jnp.int32),
            pltpu.VMEM((D * BPW,), jnp.float32),
            pltpu.VMEM((D * BPW,), jnp.float32),
            pltpu.VMEM((BPW,), jnp.float32),
            pltpu.SemaphoreType.DMA,
            pltpu.SemaphoreType.DMA,
        ],
    )
    return k(user, item, user_factors.T, item_factors.T)


# speedup vs baseline: 3.1623x; 3.1623x over previous
import jax
import jax.numpy as jnp
from jax import lax
from jax.experimental import pallas as pl
from jax.experimental.pallas import tpu as pltpu
from jax.experimental.pallas import tpu_sc as plsc

B = 16384
D = 16
NW = 32
BPW = B // NW
CHUNK = 128
NCH = BPW // CHUNK

NROWS = 1000000
PACK = 128 // D
PROWS = NROWS // PACK


def _mf_body(user_hbm, item_hbm, uf_hbm, if_hbm, out_hbm,
             uidx_v, vidx_v, urow_v, vrow_v, urows_v, vrows_v, out_v,
             sem_u, sem_v):
    c = lax.axis_index("c")
    s = lax.axis_index("s")
    wid = s * 2 + c
    base = wid * BPW

    pltpu.sync_copy(user_hbm.at[pl.ds(base, BPW)], uidx_v)
    pltpu.sync_copy(item_hbm.at[pl.ds(base, BPW)], vidx_v)

    def build_rows(g, _):
        gbase = g * 16
        urow_v[pl.ds(gbase, 16)] = uidx_v[pl.ds(gbase, 16)] >> 3
        vrow_v[pl.ds(gbase, 16)] = vidx_v[pl.ds(gbase, 16)] >> 3
        return ()

    lax.fori_loop(0, BPW // 16, build_rows, ())

    lane = lax.iota(jnp.int32, 16)

    for ch in range(NCH):
        cbase = ch * CHUNK
        sl = pl.ds(cbase, CHUNK)
        cp_u = pltpu.async_copy(uf_hbm.at[urow_v.at[sl]], urows_v, sem_u)
        cp_v = pltpu.async_copy(if_hbm.at[vrow_v.at[sl]], vrows_v, sem_v)
        cp_u.wait()
        cp_v.wait()

        def group(g, _, cbase=cbase):
            gbase = g * 16
            row = gbase + lane
            usub = (uidx_v[pl.ds(cbase + gbase, 16)] & 7) << 4
            vsub = (vidx_v[pl.ds(cbase + gbase, 16)] & 7) << 4
            acc = jnp.zeros((16,), jnp.float32)
            for d in range(D):
                u = plsc.load_gather(urows_v, [row, usub + d])
                v = plsc.load_gather(vrows_v, [row, vsub + d])
                acc = acc + u * v
            out_v[pl.ds(cbase + gbase, 16)] = acc
            return ()

        lax.fori_loop(0, CHUNK // 16, group, ())

    pltpu.sync_copy(out_v, out_hbm.at[pl.ds(base, BPW)])


def kernel(user, item, user_factors, item_factors):
    mesh = plsc.VectorSubcoreMesh(core_axis_name="c", subcore_axis_name="s")
    k = pl.kernel(
        _mf_body,
        out_type=jax.ShapeDtypeStruct((B,), jnp.float32),
        mesh=mesh,
        compiler_params=pltpu.CompilerParams(
            needs_layout_passes=False, use_tc_tiling_on_sc=True),
        scratch_types=[
            pltpu.VMEM((BPW,), jnp.int32),
            pltpu.VMEM((BPW,), jnp.int32),
            pltpu.VMEM((BPW,), jnp.int32),
            pltpu.VMEM((BPW,), jnp.int32),
            pltpu.VMEM((CHUNK, 128), jnp.float32),
            pltpu.VMEM((CHUNK, 128), jnp.float32),
            pltpu.VMEM((BPW,), jnp.float32),
            pltpu.SemaphoreType.DMA,
            pltpu.SemaphoreType.DMA,
        ],
    )
    uf128 = user_factors.reshape(PROWS, PACK * D)
    if128 = item_factors.reshape(PROWS, PACK * D)
    return k(user, item, uf128, if128)


# speedup vs baseline: 16.7110x; 5.2845x over previous
import jax
import jax.numpy as jnp
from jax import lax
from jax.experimental import pallas as pl
from jax.experimental.pallas import tpu as pltpu
from jax.experimental.pallas import tpu_sc as plsc

B = 16384
D = 16
NW = 32
BPW = B // NW
NG = BPW // 16

NROWS = 1000000


def _mf_body(user_hbm, item_hbm, uft_hbm, ift_hbm, out_hbm,
             uidx_v, vidx_v, ubuf_v, vbuf_v, uscr_v, vscr_v, out_v,
             sem_u, sem_v):
    c = lax.axis_index("c")
    s = lax.axis_index("s")
    wid = s * 2 + c
    base = wid * BPW

    pltpu.sync_copy(user_hbm.at[pl.ds(base, BPW)], uidx_v)
    pltpu.sync_copy(item_hbm.at[pl.ds(base, BPW)], vidx_v)

    lane = lax.iota(jnp.int32, 16)

    def group(g, _):
        gbase = g * 16
        u = uidx_v[pl.ds(gbase, 16)]
        v = vidx_v[pl.ds(gbase, 16)]

        copies = []
        for e in range(16):
            uoff = pl.multiple_of((u[e] >> 7) * 128, 128)
            voff = pl.multiple_of((v[e] >> 7) * 128, 128)
            copies.append(pltpu.async_copy(
                uft_hbm.at[:, pl.ds(uoff, 128)],
                ubuf_v.at[:, pl.ds(e * 128, 128)], sem_u))
            copies.append(pltpu.async_copy(
                ift_hbm.at[:, pl.ds(voff, 128)],
                vbuf_v.at[:, pl.ds(e * 128, 128)], sem_v))
        for cp in copies:
            cp.wait()

        for e in range(16):
            ucol = jnp.full((16,), (u[e] & 127) + e * 128, jnp.int32)
            vcol = jnp.full((16,), (v[e] & 127) + e * 128, jnp.int32)
            ue = plsc.load_gather(ubuf_v, [lane, ucol])
            ve = plsc.load_gather(vbuf_v, [lane, vcol])
            ecol = jnp.full((16,), e, jnp.int32)
            plsc.store_scatter(uscr_v, [lane, ecol], ue)
            plsc.store_scatter(vscr_v, [lane, ecol], ve)

        acc = jnp.zeros((16,), jnp.float32)
        for d in range(D):
            acc = acc + uscr_v[d] * vscr_v[d]
        out_v[pl.ds(gbase, 16)] = acc
        return ()

    lax.fori_loop(0, NG, group, ())

    pltpu.sync_copy(out_v, out_hbm.at[pl.ds(base, BPW)])


def kernel(user, item, user_factors, item_factors):
    mesh = plsc.VectorSubcoreMesh(core_axis_name="c", subcore_axis_name="s")
    k = pl.kernel(
        _mf_body,
        out_type=jax.ShapeDtypeStruct((B,), jnp.float32),
        mesh=mesh,
        compiler_params=pltpu.CompilerParams(
            needs_layout_passes=False, use_tc_tiling_on_sc=True),
        scratch_types=[
            pltpu.VMEM((BPW,), jnp.int32),
            pltpu.VMEM((BPW,), jnp.int32),
            pltpu.VMEM((D, 16 * 128), jnp.float32),
            pltpu.VMEM((D, 16 * 128), jnp.float32),
            pltpu.VMEM((D, 16), jnp.float32),
            pltpu.VMEM((D, 16), jnp.float32),
            pltpu.VMEM((BPW,), jnp.float32),
            pltpu.SemaphoreType.DMA,
            pltpu.SemaphoreType.DMA,
        ],
    )
    return k(user, item, user_factors.T, item_factors.T)


# speedup vs baseline: 19.7716x; 1.1831x over previous
import jax
import jax.numpy as jnp
from jax import lax
from jax.experimental import pallas as pl
from jax.experimental.pallas import tpu as pltpu
from jax.experimental.pallas import tpu_sc as plsc

B = 16384
D = 16
NW = 32
BPW = B // NW
WAVE = 8
NWAVE = BPW // WAVE

NROWS = 1000000


def _mf_body(user_hbm, item_hbm, uft_hbm, ift_hbm, out_hbm,
             uidx_v, vidx_v, ubuf_v, vbuf_v, uscr_v, vscr_v, out_v,
             sem_u0, sem_u1, sem_v0, sem_v1):
    c = lax.axis_index("c")
    s = lax.axis_index("s")
    wid = s * 2 + c
    base = wid * BPW

    pltpu.sync_copy(user_hbm.at[pl.ds(base, BPW)], uidx_v.at[pl.ds(0, BPW)])
    pltpu.sync_copy(item_hbm.at[pl.ds(base, BPW)], vidx_v.at[pl.ds(0, BPW)])

    lane = lax.iota(jnp.int32, 16)

    def fire(w, sem_u, sem_v, half):
        u = uidx_v[pl.ds(w * WAVE, 16)]
        v = vidx_v[pl.ds(w * WAVE, 16)]
        for e in range(WAVE):
            uoff = pl.multiple_of((u[e] >> 7) * 128, 128)
            voff = pl.multiple_of((v[e] >> 7) * 128, 128)
            dsl = pl.ds(half * (WAVE * 128) + e * 128, 128)
            pltpu.async_copy(uft_hbm.at[:, pl.ds(uoff, 128)],
                             ubuf_v.at[:, dsl], sem_u)
            pltpu.async_copy(ift_hbm.at[:, pl.ds(voff, 128)],
                             vbuf_v.at[:, dsl], sem_v)

    def drain(sem_u, sem_v):
        dsl = pl.ds(0, WAVE * 128)
        pltpu.make_async_copy(uft_hbm.at[:, dsl], ubuf_v.at[:, dsl],
                              sem_u).wait()
        pltpu.make_async_copy(ift_hbm.at[:, dsl], vbuf_v.at[:, dsl],
                              sem_v).wait()

    def extract(w, half):
        u = uidx_v[pl.ds(w * WAVE, 16)]
        v = vidx_v[pl.ds(w * WAVE, 16)]
        for e in range(WAVE):
            boff = half * (WAVE * 128) + e * 128
            ucol = jnp.full((16,), (u[e] & 127) + boff, jnp.int32)
            vcol = jnp.full((16,), (v[e] & 127) + boff, jnp.int32)
            ue = plsc.load_gather(ubuf_v, [lane, ucol])
            ve = plsc.load_gather(vbuf_v, [lane, vcol])
            ecol = jnp.full((16,), e + half * WAVE, jnp.int32)
            plsc.store_scatter(uscr_v, [lane, ecol], ue)
            plsc.store_scatter(vscr_v, [lane, ecol], ve)

    def mac_out(w):
        acc = jnp.zeros((16,), jnp.float32)
        for d in range(D):
            acc = acc + uscr_v[d] * vscr_v[d]
        out_v[pl.ds(((w - 1) >> 1) * 16, 16)] = acc

    def body(w, _):
        even = (w & 1) == 0

        @pl.when(jnp.logical_and(even, w < NWAVE))
        def _():
            fire(w, sem_u0, sem_v0, 0)

        @pl.when(jnp.logical_and(jnp.logical_not(even), w < NWAVE))
        def _():
            fire(w, sem_u1, sem_v1, 1)

        @pl.when(jnp.logical_and(even, w > 0))
        def _():
            drain(sem_u1, sem_v1)
            extract(w - 1, 1)
            mac_out(w)

        @pl.when(jnp.logical_and(jnp.logical_not(even), w > 0))
        def _():
            drain(sem_u0, sem_v0)
            extract(w - 1, 0)

        return ()

    lax.fori_loop(0, NWAVE + 1, body, ())

    pltpu.sync_copy(out_v, out_hbm.at[pl.ds(base, BPW)])


def kernel(user, item, user_factors, item_factors):
    mesh = plsc.VectorSubcoreMesh(core_axis_name="c", subcore_axis_name="s")
    k = pl.kernel(
        _mf_body,
        out_type=jax.ShapeDtypeStruct((B,), jnp.float32),
        mesh=mesh,
        compiler_params=pltpu.CompilerParams(
            needs_layout_passes=False, use_tc_tiling_on_sc=True),
        scratch_types=[
            pltpu.VMEM((BPW + 16,), jnp.int32),
            pltpu.VMEM((BPW + 16,), jnp.int32),
            pltpu.VMEM((D, 2 * WAVE * 128), jnp.float32),
            pltpu.VMEM((D, 2 * WAVE * 128), jnp.float32),
            pltpu.VMEM((D, 16), jnp.float32),
            pltpu.VMEM((D, 16), jnp.float32),
            pltpu.VMEM((BPW,), jnp.float32),
            pltpu.SemaphoreType.DMA,
            pltpu.SemaphoreType.DMA,
            pltpu.SemaphoreType.DMA,
            pltpu.SemaphoreType.DMA,
        ],
    )
    return k(user, item, user_factors.T, item_factors.T)
